# baseline (speedup 1.0000x reference)
import functools

import numpy as np
import jax
import jax.numpy as jnp
from jax import lax
from jax.experimental import pallas as pl
from jax.experimental.pallas import tpu as pltpu
from jax.experimental.pallas import tpu_sc as plsc

N_LEVELS = 16
F_PER_LEVEL = 2
LOG2_T = 16
TABLE_SIZE = 1 << LOG2_T
BASE_RES_ = 16
FINEST_RES_ = 512
N_DFT = 4
MLP_OUT_ = 16
N_PTS = 131072
HIDDEN_ = 64

_GROWTH = np.exp((np.log(FINEST_RES_) - np.log(BASE_RES_)) / (N_LEVELS - 1))
_RES_LIST = [float(np.floor(BASE_RES_ * _GROWTH ** l)) for l in range(N_LEVELS)]
_P1 = int(np.int32(np.uint32(2654435761)))
_P2 = int(np.int32(np.uint32(805459861)))

NC = 2
NS = 16
NW = NC * NS
P_PER_W = N_PTS // NW
CHUNK = 16
N_CHUNKS = P_PER_W // CHUNK


def _sc_embed_body(xr_hbm, yr_hbm, zr_hbm, res_hbm, tab_hbm, out_hbm,
                   xb_v, yb_v, zb_v, res_v,
                   idx0_v, idx1_v, w0_v, w1_v, rows0_v, rows1_v, outc_v, sem):
    cid = lax.axis_index("c")
    sid = lax.axis_index("s")
    wid = sid * NC + cid
    wbase = wid * P_PER_W

    pltpu.sync_copy(xr_hbm.at[pl.ds(wbase, P_PER_W)], xb_v)
    pltpu.sync_copy(yr_hbm.at[pl.ds(wbase, P_PER_W)], yb_v)
    pltpu.sync_copy(zr_hbm.at[pl.ds(wbase, P_PER_W)], zb_v)
    pltpu.sync_copy(res_hbm, res_v)

    lanes = lax.iota(jnp.int32, 16)
    lanes8 = lanes * 8

    def phase_a(ci, idx_v, w_v):
        base = ci * CHUNK
        px = xb_v[pl.ds(base, CHUNK)]
        py = yb_v[pl.ds(base, CHUNK)]
        pz = zb_v[pl.ds(base, CHUNK)]

        def lvl_a(l, c2):
            lsplat = jnp.full((16,), l, jnp.int32)
            r = plsc.load_gather(res_v, [lsplat])
            xs = px * r
            ys = py * r
            zs = pz * r
            xi = xs.astype(jnp.int32)
            yi = ys.astype(jnp.int32)
            zi = zs.astype(jnp.int32)
            wx1 = xs - xi.astype(jnp.float32)
            wy1 = ys - yi.astype(jnp.float32)
            wz1 = zs - zi.astype(jnp.float32)
            wx0 = 1.0 - wx1
            wy0 = 1.0 - wy1
            wz0 = 1.0 - wz1
            hx = (xi, xi + 1)
            hy = (yi * _P1, yi * _P1 + _P1)
            hz = (zi * _P2, zi * _P2 + _P2)
            wyz = (wy0 * wz0, wy0 * wz1, wy1 * wz0, wy1 * wz1)
            wx = (wx0, wx1)
            lbase = l * TABLE_SIZE
            l128 = l * 128
            for o in range(8):
                i, j, k = (o >> 2) & 1, (o >> 1) & 1, o & 1
                h = ((hx[i] ^ hy[j] ^ hz[k]) & 0xFFFF) + lbase
                col = lanes + (o * 16)
                rowv = col + l128
                plsc.store_scatter(idx_v, [lsplat, col], h)
                plsc.store_scatter(w_v, [rowv], wx[i] * wyz[2 * j + k])
            return c2
        lax.fori_loop(0, N_LEVELS, lvl_a, 0, unroll=False)

    def fire(idx_v, rows_v):
        def f_(l, c2):
            pltpu.async_copy(tab_hbm.at[idx_v.at[l]],
                             rows_v.at[pl.ds(l * 128, 128), :], sem)
            return c2
        lax.fori_loop(0, N_LEVELS, f_, 0, unroll=False)

    def drain(idx_v, rows_v):
        def d_(l, c2):
            pltpu.make_async_copy(tab_hbm.at[idx_v.at[l]],
                                  rows_v.at[pl.ds(l * 128, 128), :], sem).wait()
            return c2
        lax.fori_loop(0, N_LEVELS, d_, 0, unroll=False)

    fsplats = [jnp.full((16,), f, jnp.int32) for f in range(8)]

    def phase_b(ci, w_v, rows_v):
        def lvl_b(l, c2):
            l128 = l * 128
            lb8 = l * 8
            acc = [jnp.zeros((16,), jnp.float32) for _ in range(8)]
            for o in range(8):
                rowv = lanes + (l128 + o * 16)
                wv = plsc.load_gather(w_v, [rowv])
                for f in range(8):
                    v = plsc.load_gather(rows_v, [rowv, fsplats[f]])
                    acc[f] = acc[f] + wv * v
            for f in range(8):
                plsc.store_scatter(outc_v, [lanes, jnp.full((16,), lb8 + f, jnp.int32)],
                                   acc[f])
            return c2
        lax.fori_loop(0, N_LEVELS, lvl_b, 0, unroll=2)

        pltpu.sync_copy(outc_v, out_hbm.at[pl.ds(wbase + ci * CHUNK, CHUNK), :])

    phase_a(0, idx0_v, w0_v)
    fire(idx0_v, rows0_v)

    def body2(j, carry):
        ci = j * 2
        phase_a(ci + 1, idx1_v, w1_v)
        fire(idx1_v, rows1_v)
        drain(idx0_v, rows0_v)
        phase_b(ci, w0_v, rows0_v)

        @pl.when(j < N_CHUNKS // 2 - 1)
        def _():
            phase_a(ci + 2, idx0_v, w0_v)
            fire(idx0_v, rows0_v)

        drain(idx1_v, rows1_v)
        phase_b(ci + 1, w1_v, rows1_v)
        return carry

    lax.fori_loop(0, N_CHUNKS // 2, body2, 0, unroll=False)


@functools.cache
def _build_sc_embed():
    mesh = plsc.VectorSubcoreMesh(core_axis_name="c", subcore_axis_name="s")
    return pl.kernel(
        _sc_embed_body,
        out_type=jax.ShapeDtypeStruct((N_PTS, N_LEVELS * N_DFT * F_PER_LEVEL),
                                      jnp.float32),
        mesh=mesh,
        compiler_params=pltpu.CompilerParams(needs_layout_passes=False,
                                             use_tc_tiling_on_sc=False),
        scratch_types=[
            pltpu.VMEM((P_PER_W,), jnp.float32),
            pltpu.VMEM((P_PER_W,), jnp.float32),
            pltpu.VMEM((P_PER_W,), jnp.float32),
            pltpu.VMEM((N_LEVELS,), jnp.float32),
            pltpu.VMEM((N_LEVELS, 8 * CHUNK), jnp.int32),
            pltpu.VMEM((N_LEVELS, 8 * CHUNK), jnp.int32),
            pltpu.VMEM((N_LEVELS * 8 * CHUNK,), jnp.float32),
            pltpu.VMEM((N_LEVELS * 8 * CHUNK,), jnp.float32),
            pltpu.VMEM((N_LEVELS * 8 * CHUNK, N_DFT * F_PER_LEVEL), jnp.float32),
            pltpu.VMEM((N_LEVELS * 8 * CHUNK, N_DFT * F_PER_LEVEL), jnp.float32),
            pltpu.VMEM((CHUNK, N_LEVELS * N_DFT * F_PER_LEVEL), jnp.float32),
            pltpu.SemaphoreType.DMA,
        ],
    )


def _mlp_body(t_ref, g_ref, w1_ref, b1_ref, w2_ref, b2_ref, w3_ref, b3_ref, o_ref):
    tb = t_ref[...]
    c1 = jnp.cos(np.float32(np.pi) * tb)
    c2 = 2.0 * c1 * c1 - 1.0
    c3 = 2.0 * c2 * c1 - c1
    col = lax.broadcasted_iota(jnp.int32, (1, 128), 1)
    kk = (col % 8) // 2
    basis = jnp.where(kk == 0, 1.0,
                      jnp.where(kk == 1, c1, jnp.where(kk == 2, c2, c3)))
    g = g_ref[...] * basis
    srow = lax.broadcasted_iota(jnp.int32, (128, 32), 0)
    scol = lax.broadcasted_iota(jnp.int32, (128, 32), 1)
    sel = (scol == (srow // 8) * 2 + srow % 2).astype(jnp.float32)
    h32 = jnp.dot(g, sel, preferred_element_type=jnp.float32)
    h = jnp.maximum(jnp.dot(h32, w1_ref[...], preferred_element_type=jnp.float32)
                    + b1_ref[...], 0.0)
    h = jnp.maximum(jnp.dot(h, w2_ref[...], preferred_element_type=jnp.float32)
                    + b2_ref[...], 0.0)
    o_ref[...] = (jnp.dot(h, w3_ref[...], preferred_element_type=jnp.float32)
                  + b3_ref[...])


_BN = 1024


@functools.cache
def _build_mlp():
    d_in = N_LEVELS * N_DFT * F_PER_LEVEL
    return pl.pallas_call(
        _mlp_body,
        grid=(N_PTS // _BN,),
        in_specs=[
            pl.BlockSpec((_BN, 1), lambda i: (i, 0)),
            pl.BlockSpec((_BN, d_in), lambda i: (i, 0)),
            pl.BlockSpec((2 * N_LEVELS, HIDDEN_), lambda i: (0, 0)),
            pl.BlockSpec((1, HIDDEN_), lambda i: (0, 0)),
            pl.BlockSpec((HIDDEN_, HIDDEN_), lambda i: (0, 0)),
            pl.BlockSpec((1, HIDDEN_), lambda i: (0, 0)),
            pl.BlockSpec((HIDDEN_, MLP_OUT_), lambda i: (0, 0)),
            pl.BlockSpec((1, MLP_OUT_), lambda i: (0, 0)),
        ],
        out_specs=pl.BlockSpec((_BN, MLP_OUT_), lambda i: (i, 0)),
        out_shape=jax.ShapeDtypeStruct((N_PTS, MLP_OUT_), jnp.float32),
    )


def kernel(x, t, tables, W1, b1, W2, b2, W3, b3):
    tab_flat = tables.reshape(N_LEVELS * TABLE_SIZE, N_DFT * F_PER_LEVEL)
    res = jnp.asarray(_RES_LIST, jnp.float32)
    feats = _build_sc_embed()(x[:, 0], x[:, 1], x[:, 2], res, tab_flat)
    return _build_mlp()(t[:, None], feats, W1, b1[None], W2, b2[None], W3, b3[None])


# speedup vs baseline: 1.1418x; 1.1418x over previous
import functools

import numpy as np
import jax
import jax.numpy as jnp
from jax import lax
from jax.experimental import pallas as pl
from jax.experimental.pallas import tpu as pltpu
from jax.experimental.pallas import tpu_sc as plsc

N_LEVELS = 16
F_PER_LEVEL = 2
LOG2_T = 16
TABLE_SIZE = 1 << LOG2_T
BASE_RES_ = 16
FINEST_RES_ = 512
N_DFT = 4
MLP_OUT_ = 16
N_PTS = 131072
HIDDEN_ = 64

_GROWTH = np.exp((np.log(FINEST_RES_) - np.log(BASE_RES_)) / (N_LEVELS - 1))
_RES_LIST = [float(np.floor(BASE_RES_ * _GROWTH ** l)) for l in range(N_LEVELS)]
_P1 = int(np.int32(np.uint32(2654435761)))
_P2 = int(np.int32(np.uint32(805459861)))

NC = 2
NS = 16
NW = NC * NS
P_PER_W = N_PTS // NW
CHUNK = 16
N_CHUNKS = P_PER_W // CHUNK


def _sc_embed_body(xr_hbm, yr_hbm, zr_hbm, c1_hbm, c2_hbm, c3_hbm, res_hbm,
                   tab_hbm, out_hbm,
                   xb_v, yb_v, zb_v, c1b_v, c2b_v, c3b_v, res_v,
                   idx0_v, idx1_v, w0_v, w1_v, rows0_v, rows1_v, outc_v, sem):
    cid = lax.axis_index("c")
    sid = lax.axis_index("s")
    wid = sid * NC + cid
    wbase = wid * P_PER_W

    pltpu.sync_copy(xr_hbm.at[pl.ds(wbase, P_PER_W)], xb_v)
    pltpu.sync_copy(yr_hbm.at[pl.ds(wbase, P_PER_W)], yb_v)
    pltpu.sync_copy(zr_hbm.at[pl.ds(wbase, P_PER_W)], zb_v)
    pltpu.sync_copy(c1_hbm.at[pl.ds(wbase, P_PER_W)], c1b_v)
    pltpu.sync_copy(c2_hbm.at[pl.ds(wbase, P_PER_W)], c2b_v)
    pltpu.sync_copy(c3_hbm.at[pl.ds(wbase, P_PER_W)], c3b_v)
    pltpu.sync_copy(res_hbm, res_v)

    lanes = lax.iota(jnp.int32, 16)
    lanes8 = lanes * 8

    def phase_a(ci, idx_v, w_v):
        base = ci * CHUNK
        px = xb_v[pl.ds(base, CHUNK)]
        py = yb_v[pl.ds(base, CHUNK)]
        pz = zb_v[pl.ds(base, CHUNK)]

        def lvl_a(l, c2):
            lsplat = jnp.full((16,), l, jnp.int32)
            r = plsc.load_gather(res_v, [lsplat])
            xs = px * r
            ys = py * r
            zs = pz * r
            xi = xs.astype(jnp.int32)
            yi = ys.astype(jnp.int32)
            zi = zs.astype(jnp.int32)
            wx1 = xs - xi.astype(jnp.float32)
            wy1 = ys - yi.astype(jnp.float32)
            wz1 = zs - zi.astype(jnp.float32)
            wx0 = 1.0 - wx1
            wy0 = 1.0 - wy1
            wz0 = 1.0 - wz1
            hx = (xi, xi + 1)
            hy = (yi * _P1, yi * _P1 + _P1)
            hz = (zi * _P2, zi * _P2 + _P2)
            wyz = (wy0 * wz0, wy0 * wz1, wy1 * wz0, wy1 * wz1)
            wx = (wx0, wx1)
            lbase = l * TABLE_SIZE
            l128 = l * 128
            for o in range(8):
                i, j, k = (o >> 2) & 1, (o >> 1) & 1, o & 1
                h = ((hx[i] ^ hy[j] ^ hz[k]) & 0xFFFF) + lbase
                col = lanes + (o * 16)
                rowv = col + l128
                plsc.store_scatter(idx_v, [lsplat, col], h)
                plsc.store_scatter(w_v, [rowv], wx[i] * wyz[2 * j + k])
            return c2
        lax.fori_loop(0, N_LEVELS, lvl_a, 0, unroll=False)

    def fire(idx_v, rows_v):
        def f_(l, c2):
            pltpu.async_copy(tab_hbm.at[idx_v.at[l]],
                             rows_v.at[pl.ds(l * 128, 128), :], sem)
            return c2
        lax.fori_loop(0, N_LEVELS, f_, 0, unroll=False)

    def drain(idx_v, rows_v):
        def d_(l, c2):
            pltpu.make_async_copy(tab_hbm.at[idx_v.at[l]],
                                  rows_v.at[pl.ds(l * 128, 128), :], sem).wait()
            return c2
        lax.fori_loop(0, N_LEVELS, d_, 0, unroll=False)

    fsplats = [jnp.full((16,), f, jnp.int32) for f in range(8)]

    def phase_b(ci, w_v, rows_v):
        base = ci * CHUNK
        cks = (None, c1b_v[pl.ds(base, CHUNK)], c2b_v[pl.ds(base, CHUNK)],
               c3b_v[pl.ds(base, CHUNK)])

        def lvl_b(l, c2):
            l128 = l * 128
            lb8 = l * 8
            acc = [jnp.zeros((16,), jnp.float32) for _ in range(8)]
            for o in range(8):
                rowv = lanes + (l128 + o * 16)
                wv = plsc.load_gather(w_v, [rowv])
                for f in range(8):
                    v = plsc.load_gather(rows_v, [rowv, fsplats[f]])
                    acc[f] = acc[f] + wv * v
            for f in range(8):
                val = acc[f] if cks[f // 2] is None else acc[f] * cks[f // 2]
                plsc.store_scatter(outc_v, [lanes, jnp.full((16,), lb8 + f, jnp.int32)],
                                   val)
            return c2
        lax.fori_loop(0, N_LEVELS, lvl_b, 0, unroll=False)

        pltpu.sync_copy(outc_v, out_hbm.at[pl.ds(wbase + ci * CHUNK, CHUNK), :])

    phase_a(0, idx0_v, w0_v)
    fire(idx0_v, rows0_v)

    def body2(j, carry):
        ci = j * 2
        phase_a(ci + 1, idx1_v, w1_v)
        fire(idx1_v, rows1_v)
        drain(idx0_v, rows0_v)
        phase_b(ci, w0_v, rows0_v)

        @pl.when(j < N_CHUNKS // 2 - 1)
        def _():
            phase_a(ci + 2, idx0_v, w0_v)
            fire(idx0_v, rows0_v)

        drain(idx1_v, rows1_v)
        phase_b(ci + 1, w1_v, rows1_v)
        return carry

    lax.fori_loop(0, N_CHUNKS // 2, body2, 0, unroll=False)


@functools.cache
def _build_sc_embed():
    mesh = plsc.VectorSubcoreMesh(core_axis_name="c", subcore_axis_name="s")
    return pl.kernel(
        _sc_embed_body,
        out_type=jax.ShapeDtypeStruct((N_PTS, N_LEVELS * N_DFT * F_PER_LEVEL),
                                      jnp.float32),
        mesh=mesh,
        compiler_params=pltpu.CompilerParams(needs_layout_passes=False,
                                             use_tc_tiling_on_sc=False),
        scratch_types=[
            pltpu.VMEM((P_PER_W,), jnp.float32),
            pltpu.VMEM((P_PER_W,), jnp.float32),
            pltpu.VMEM((P_PER_W,), jnp.float32),
            pltpu.VMEM((P_PER_W,), jnp.float32),
            pltpu.VMEM((P_PER_W,), jnp.float32),
            pltpu.VMEM((P_PER_W,), jnp.float32),
            pltpu.VMEM((N_LEVELS,), jnp.float32),
            pltpu.VMEM((N_LEVELS, 8 * CHUNK), jnp.int32),
            pltpu.VMEM((N_LEVELS, 8 * CHUNK), jnp.int32),
            pltpu.VMEM((N_LEVELS * 8 * CHUNK,), jnp.float32),
            pltpu.VMEM((N_LEVELS * 8 * CHUNK,), jnp.float32),
            pltpu.VMEM((N_LEVELS * 8 * CHUNK, N_DFT * F_PER_LEVEL), jnp.float32),
            pltpu.VMEM((N_LEVELS * 8 * CHUNK, N_DFT * F_PER_LEVEL), jnp.float32),
            pltpu.VMEM((CHUNK, N_LEVELS * N_DFT * F_PER_LEVEL), jnp.float32),
            pltpu.SemaphoreType.DMA,
        ],
    )


_BT = 8192


def _basis_body(t_ref, c1_ref, c2_ref, c3_ref):
    t = t_ref[...]
    c1 = jnp.cos(np.float32(np.pi) * t)
    c2 = 2.0 * c1 * c1 - 1.0
    c3 = 2.0 * c2 * c1 - c1
    c1_ref[...] = c1
    c2_ref[...] = c2
    c3_ref[...] = c3


@functools.cache
def _build_basis():
    return pl.pallas_call(
        _basis_body,
        grid=(N_PTS // _BT,),
        in_specs=[pl.BlockSpec((_BT,), lambda i: (i,))],
        out_specs=[pl.BlockSpec((_BT,), lambda i: (i,))] * 3,
        out_shape=[jax.ShapeDtypeStruct((N_PTS,), jnp.float32)] * 3,
    )


def _mlp_body(g_ref, w1_ref, b1_ref, w2_ref, b2_ref, w3_ref, b3_ref, o_ref):
    g = g_ref[...]
    srow = lax.broadcasted_iota(jnp.int32, (128, 32), 0)
    scol = lax.broadcasted_iota(jnp.int32, (128, 32), 1)
    sel = (scol == (srow // 8) * 2 + srow % 2).astype(jnp.float32)
    h32 = jnp.dot(g, sel, preferred_element_type=jnp.float32)
    h = jnp.maximum(jnp.dot(h32, w1_ref[...], preferred_element_type=jnp.float32)
                    + b1_ref[...], 0.0)
    h = jnp.maximum(jnp.dot(h, w2_ref[...], preferred_element_type=jnp.float32)
                    + b2_ref[...], 0.0)
    o_ref[...] = (jnp.dot(h, w3_ref[...], preferred_element_type=jnp.float32)
                  + b3_ref[...])


_BN = 1024


@functools.cache
def _build_mlp():
    d_in = N_LEVELS * N_DFT * F_PER_LEVEL
    return pl.pallas_call(
        _mlp_body,
        grid=(N_PTS // _BN,),
        in_specs=[
            pl.BlockSpec((_BN, d_in), lambda i: (i, 0)),
            pl.BlockSpec((2 * N_LEVELS, HIDDEN_), lambda i: (0, 0)),
            pl.BlockSpec((1, HIDDEN_), lambda i: (0, 0)),
            pl.BlockSpec((HIDDEN_, HIDDEN_), lambda i: (0, 0)),
            pl.BlockSpec((1, HIDDEN_), lambda i: (0, 0)),
            pl.BlockSpec((HIDDEN_, MLP_OUT_), lambda i: (0, 0)),
            pl.BlockSpec((1, MLP_OUT_), lambda i: (0, 0)),
        ],
        out_specs=pl.BlockSpec((_BN, MLP_OUT_), lambda i: (i, 0)),
        out_shape=jax.ShapeDtypeStruct((N_PTS, MLP_OUT_), jnp.float32),
    )


def kernel(x, t, tables, W1, b1, W2, b2, W3, b3):
    tab_flat = tables.reshape(N_LEVELS * TABLE_SIZE, N_DFT * F_PER_LEVEL)
    res = jnp.asarray(_RES_LIST, jnp.float32)
    c1, c2, c3 = _build_basis()(t)
    feats = _build_sc_embed()(x[:, 0], x[:, 1], x[:, 2], c1, c2, c3, res, tab_flat)
    return _build_mlp()(feats, W1, b1[None], W2, b2[None], W3, b3[None])


# speedup vs baseline: 1.1735x; 1.0278x over previous
import functools

import numpy as np
import jax
import jax.numpy as jnp
from jax import lax
from jax.experimental import pallas as pl
from jax.experimental.pallas import tpu as pltpu
from jax.experimental.pallas import tpu_sc as plsc

N_LEVELS = 16
F_PER_LEVEL = 2
LOG2_T = 16
TABLE_SIZE = 1 << LOG2_T
BASE_RES_ = 16
FINEST_RES_ = 512
N_DFT = 4
MLP_OUT_ = 16
N_PTS = 131072
HIDDEN_ = 64

_GROWTH = np.exp((np.log(FINEST_RES_) - np.log(BASE_RES_)) / (N_LEVELS - 1))
_RES_LIST = [float(np.floor(BASE_RES_ * _GROWTH ** l)) for l in range(N_LEVELS)]
_P1 = int(np.int32(np.uint32(2654435761)))
_P2 = int(np.int32(np.uint32(805459861)))

NC = 2
NS = 16
NW = NC * NS
P_PER_W = N_PTS // NW
CHUNK = 16
N_CHUNKS = P_PER_W // CHUNK


def _sc_embed_body(xr_hbm, yr_hbm, zr_hbm, c1_hbm, c2_hbm, c3_hbm, res_hbm,
                   tab_hbm, out_hbm,
                   xb_v, yb_v, zb_v, c1b_v, c2b_v, c3b_v, res_v,
                   idx0_v, idx1_v, w0_v, w1_v, rows0_v, rows1_v, outc_v, sem):
    cid = lax.axis_index("c")
    sid = lax.axis_index("s")
    wid = sid * NC + cid
    wbase = wid * P_PER_W

    pltpu.sync_copy(xr_hbm.at[pl.ds(wbase, P_PER_W)], xb_v)
    pltpu.sync_copy(yr_hbm.at[pl.ds(wbase, P_PER_W)], yb_v)
    pltpu.sync_copy(zr_hbm.at[pl.ds(wbase, P_PER_W)], zb_v)
    pltpu.sync_copy(c1_hbm.at[pl.ds(wbase, P_PER_W)], c1b_v)
    pltpu.sync_copy(c2_hbm.at[pl.ds(wbase, P_PER_W)], c2b_v)
    pltpu.sync_copy(c3_hbm.at[pl.ds(wbase, P_PER_W)], c3b_v)
    pltpu.sync_copy(res_hbm, res_v)

    lanes = lax.iota(jnp.int32, 16)
    lanes8 = lanes * 8

    def phase_a(ci, idx_v, w_v):
        base = ci * CHUNK
        px = xb_v[pl.ds(base, CHUNK)]
        py = yb_v[pl.ds(base, CHUNK)]
        pz = zb_v[pl.ds(base, CHUNK)]

        def lvl_a(l, c2):
            lsplat = jnp.full((16,), l, jnp.int32)
            r = plsc.load_gather(res_v, [lsplat])
            xs = px * r
            ys = py * r
            zs = pz * r
            xi = xs.astype(jnp.int32)
            yi = ys.astype(jnp.int32)
            zi = zs.astype(jnp.int32)
            wx1 = xs - xi.astype(jnp.float32)
            wy1 = ys - yi.astype(jnp.float32)
            wz1 = zs - zi.astype(jnp.float32)
            wx0 = 1.0 - wx1
            wy0 = 1.0 - wy1
            wz0 = 1.0 - wz1
            hx = (xi, xi + 1)
            hy = (yi * _P1, yi * _P1 + _P1)
            hz = (zi * _P2, zi * _P2 + _P2)
            wyz = (wy0 * wz0, wy0 * wz1, wy1 * wz0, wy1 * wz1)
            wx = (wx0, wx1)
            lbase = l * TABLE_SIZE
            l128 = l * 128
            for o in range(8):
                i, j, k = (o >> 2) & 1, (o >> 1) & 1, o & 1
                h = ((hx[i] ^ hy[j] ^ hz[k]) & 0xFFFF) + lbase
                rowv = lanes + (l128 + o * 16)
                plsc.store_scatter(idx_v, [rowv], h)
                plsc.store_scatter(w_v, [rowv], wx[i] * wyz[2 * j + k])
            return c2
        lax.fori_loop(0, N_LEVELS, lvl_a, 0, unroll=False)

    def fire(idx_v, rows_v):
        pltpu.async_copy(tab_hbm.at[idx_v], rows_v, sem)

    def drain(idx_v, rows_v):
        pltpu.make_async_copy(tab_hbm.at[idx_v], rows_v, sem).wait()

    fsplats = [jnp.full((16,), f, jnp.int32) for f in range(8)]

    def phase_b(ci, w_v, rows_v):
        base = ci * CHUNK
        cks = (None, c1b_v[pl.ds(base, CHUNK)], c2b_v[pl.ds(base, CHUNK)],
               c3b_v[pl.ds(base, CHUNK)])

        def lvl_b(l, c2):
            l128 = l * 128
            lb8 = l * 8
            acc = [jnp.zeros((16,), jnp.float32) for _ in range(8)]
            for o in range(8):
                rowv = lanes + (l128 + o * 16)
                wv = plsc.load_gather(w_v, [rowv])
                for f in range(8):
                    v = plsc.load_gather(rows_v, [rowv, fsplats[f]])
                    acc[f] = acc[f] + wv * v
            for f in range(8):
                val = acc[f] if cks[f // 2] is None else acc[f] * cks[f // 2]
                plsc.store_scatter(outc_v, [lanes, jnp.full((16,), lb8 + f, jnp.int32)],
                                   val)
            return c2
        lax.fori_loop(0, N_LEVELS, lvl_b, 0, unroll=False)

        pltpu.sync_copy(outc_v, out_hbm.at[pl.ds(wbase + ci * CHUNK, CHUNK), :])

    phase_a(0, idx0_v, w0_v)
    fire(idx0_v, rows0_v)

    def body2(j, carry):
        ci = j * 2
        phase_a(ci + 1, idx1_v, w1_v)
        fire(idx1_v, rows1_v)
        drain(idx0_v, rows0_v)
        phase_b(ci, w0_v, rows0_v)

        @pl.when(j < N_CHUNKS // 2 - 1)
        def _():
            phase_a(ci + 2, idx0_v, w0_v)
            fire(idx0_v, rows0_v)

        drain(idx1_v, rows1_v)
        phase_b(ci + 1, w1_v, rows1_v)
        return carry

    lax.fori_loop(0, N_CHUNKS // 2, body2, 0, unroll=False)


@functools.cache
def _build_sc_embed():
    mesh = plsc.VectorSubcoreMesh(core_axis_name="c", subcore_axis_name="s")
    return pl.kernel(
        _sc_embed_body,
        out_type=jax.ShapeDtypeStruct((N_PTS, N_LEVELS * N_DFT * F_PER_LEVEL),
                                      jnp.float32),
        mesh=mesh,
        compiler_params=pltpu.CompilerParams(needs_layout_passes=False,
                                             use_tc_tiling_on_sc=False),
        scratch_types=[
            pltpu.VMEM((P_PER_W,), jnp.float32),
            pltpu.VMEM((P_PER_W,), jnp.float32),
            pltpu.VMEM((P_PER_W,), jnp.float32),
            pltpu.VMEM((P_PER_W,), jnp.float32),
            pltpu.VMEM((P_PER_W,), jnp.float32),
            pltpu.VMEM((P_PER_W,), jnp.float32),
            pltpu.VMEM((N_LEVELS,), jnp.float32),
            pltpu.VMEM((N_LEVELS * 8 * CHUNK,), jnp.int32),
            pltpu.VMEM((N_LEVELS * 8 * CHUNK,), jnp.int32),
            pltpu.VMEM((N_LEVELS * 8 * CHUNK,), jnp.float32),
            pltpu.VMEM((N_LEVELS * 8 * CHUNK,), jnp.float32),
            pltpu.VMEM((N_LEVELS * 8 * CHUNK, N_DFT * F_PER_LEVEL), jnp.float32),
            pltpu.VMEM((N_LEVELS * 8 * CHUNK, N_DFT * F_PER_LEVEL), jnp.float32),
            pltpu.VMEM((CHUNK, N_LEVELS * N_DFT * F_PER_LEVEL), jnp.float32),
            pltpu.SemaphoreType.DMA,
        ],
    )


_BT = 8192


def _basis_body(t_ref, c1_ref, c2_ref, c3_ref):
    t = t_ref[...]
    c1 = jnp.cos(np.float32(np.pi) * t)
    c2 = 2.0 * c1 * c1 - 1.0
    c3 = 2.0 * c2 * c1 - c1
    c1_ref[...] = c1
    c2_ref[...] = c2
    c3_ref[...] = c3


@functools.cache
def _build_basis():
    return pl.pallas_call(
        _basis_body,
        grid=(N_PTS // _BT,),
        in_specs=[pl.BlockSpec((_BT,), lambda i: (i,))],
        out_specs=[pl.BlockSpec((_BT,), lambda i: (i,))] * 3,
        out_shape=[jax.ShapeDtypeStruct((N_PTS,), jnp.float32)] * 3,
    )


def _mlp_body(g_ref, w1_ref, b1_ref, w2_ref, b2_ref, w3_ref, b3_ref, o_ref):
    g = g_ref[...]
    srow = lax.broadcasted_iota(jnp.int32, (128, 32), 0)
    scol = lax.broadcasted_iota(jnp.int32, (128, 32), 1)
    sel = (scol == (srow // 8) * 2 + srow % 2).astype(jnp.float32)
    h32 = jnp.dot(g, sel, preferred_element_type=jnp.float32)
    h = jnp.maximum(jnp.dot(h32, w1_ref[...], preferred_element_type=jnp.float32)
                    + b1_ref[...], 0.0)
    h = jnp.maximum(jnp.dot(h, w2_ref[...], preferred_element_type=jnp.float32)
                    + b2_ref[...], 0.0)
    o_ref[...] = (jnp.dot(h, w3_ref[...], preferred_element_type=jnp.float32)
                  + b3_ref[...])


_BN = 1024


@functools.cache
def _build_mlp():
    d_in = N_LEVELS * N_DFT * F_PER_LEVEL
    return pl.pallas_call(
        _mlp_body,
        grid=(N_PTS // _BN,),
        in_specs=[
            pl.BlockSpec((_BN, d_in), lambda i: (i, 0)),
            pl.BlockSpec((2 * N_LEVELS, HIDDEN_), lambda i: (0, 0)),
            pl.BlockSpec((1, HIDDEN_), lambda i: (0, 0)),
            pl.BlockSpec((HIDDEN_, HIDDEN_), lambda i: (0, 0)),
            pl.BlockSpec((1, HIDDEN_), lambda i: (0, 0)),
            pl.BlockSpec((HIDDEN_, MLP_OUT_), lambda i: (0, 0)),
            pl.BlockSpec((1, MLP_OUT_), lambda i: (0, 0)),
        ],
        out_specs=pl.BlockSpec((_BN, MLP_OUT_), lambda i: (i, 0)),
        out_shape=jax.ShapeDtypeStruct((N_PTS, MLP_OUT_), jnp.float32),
    )


def kernel(x, t, tables, W1, b1, W2, b2, W3, b3):
    tab_flat = tables.reshape(N_LEVELS * TABLE_SIZE, N_DFT * F_PER_LEVEL)
    res = jnp.asarray(_RES_LIST, jnp.float32)
    c1, c2, c3 = _build_basis()(t)
    feats = _build_sc_embed()(x[:, 0], x[:, 1], x[:, 2], c1, c2, c3, res, tab_flat)
    return _build_mlp()(feats, W1, b1[None], W2, b2[None], W3, b3[None])


# speedup vs baseline: 1.1781x; 1.0039x over previous
import functools

import numpy as np
import jax
import jax.numpy as jnp
from jax import lax
from jax.experimental import pallas as pl
from jax.experimental.pallas import tpu as pltpu
from jax.experimental.pallas import tpu_sc as plsc

N_LEVELS = 16
F_PER_LEVEL = 2
LOG2_T = 16
TABLE_SIZE = 1 << LOG2_T
BASE_RES_ = 16
FINEST_RES_ = 512
N_DFT = 4
MLP_OUT_ = 16
N_PTS = 131072
HIDDEN_ = 64

_GROWTH = np.exp((np.log(FINEST_RES_) - np.log(BASE_RES_)) / (N_LEVELS - 1))
_RES_LIST = [float(np.floor(BASE_RES_ * _GROWTH ** l)) for l in range(N_LEVELS)]
_P1 = int(np.int32(np.uint32(2654435761)))
_P2 = int(np.int32(np.uint32(805459861)))

NC = 2
NS = 16
NW = NC * NS
P_PER_W = N_PTS // NW
CHUNK = 16
N_CHUNKS = P_PER_W // CHUNK


def _sc_embed_body(xr_hbm, yr_hbm, zr_hbm, c1_hbm, c2_hbm, c3_hbm, res_hbm,
                   tab_hbm, out_hbm,
                   xb_v, yb_v, zb_v, c1b_v, c2b_v, c3b_v, res_v,
                   idx0_v, idx1_v, w0_v, w1_v, rows0_v, rows1_v, outc_v, sem):
    cid = lax.axis_index("c")
    sid = lax.axis_index("s")
    wid = sid * NC + cid
    wbase = wid * P_PER_W

    pltpu.sync_copy(xr_hbm.at[pl.ds(wbase, P_PER_W)], xb_v)
    pltpu.sync_copy(yr_hbm.at[pl.ds(wbase, P_PER_W)], yb_v)
    pltpu.sync_copy(zr_hbm.at[pl.ds(wbase, P_PER_W)], zb_v)
    pltpu.sync_copy(c1_hbm.at[pl.ds(wbase, P_PER_W)], c1b_v)
    pltpu.sync_copy(c2_hbm.at[pl.ds(wbase, P_PER_W)], c2b_v)
    pltpu.sync_copy(c3_hbm.at[pl.ds(wbase, P_PER_W)], c3b_v)
    pltpu.sync_copy(res_hbm, res_v)

    lanes = lax.iota(jnp.int32, 16)
    lanes8 = lanes * 8

    def phase_a(ci, idx_v, w_v):
        base = ci * CHUNK
        px = xb_v[pl.ds(base, CHUNK)]
        py = yb_v[pl.ds(base, CHUNK)]
        pz = zb_v[pl.ds(base, CHUNK)]

        def lvl_a(l, c2):
            lsplat = jnp.full((16,), l, jnp.int32)
            r = plsc.load_gather(res_v, [lsplat])
            xs = px * r
            ys = py * r
            zs = pz * r
            xi = xs.astype(jnp.int32)
            yi = ys.astype(jnp.int32)
            zi = zs.astype(jnp.int32)
            wx1 = xs - xi.astype(jnp.float32)
            wy1 = ys - yi.astype(jnp.float32)
            wz1 = zs - zi.astype(jnp.float32)
            wx0 = 1.0 - wx1
            wy0 = 1.0 - wy1
            wz0 = 1.0 - wz1
            hx = (xi, xi + 1)
            hy = (yi * _P1, yi * _P1 + _P1)
            hz = (zi * _P2, zi * _P2 + _P2)
            wyz = (wy0 * wz0, wy0 * wz1, wy1 * wz0, wy1 * wz1)
            wx = (wx0, wx1)
            lbase = l * TABLE_SIZE
            l128 = l * 128
            for o in range(8):
                i, j, k = (o >> 2) & 1, (o >> 1) & 1, o & 1
                h = ((hx[i] ^ hy[j] ^ hz[k]) & 0xFFFF) + lbase
                idx_v[pl.ds(l128 + o * 16, 16)] = h
                w_v[pl.ds(l128 + o * 16, 16)] = wx[i] * wyz[2 * j + k]
            return c2
        lax.fori_loop(0, N_LEVELS, lvl_a, 0, unroll=False)

    def fire(idx_v, rows_v):
        pltpu.async_copy(tab_hbm.at[idx_v], rows_v, sem)

    def drain(idx_v, rows_v):
        pltpu.make_async_copy(tab_hbm.at[idx_v], rows_v, sem).wait()

    fsplats = [jnp.full((16,), f, jnp.int32) for f in range(8)]

    def phase_b(ci, w_v, rows_v):
        base = ci * CHUNK
        cks = (None, c1b_v[pl.ds(base, CHUNK)], c2b_v[pl.ds(base, CHUNK)],
               c3b_v[pl.ds(base, CHUNK)])

        def lvl_b(l, c2):
            l128 = l * 128
            lb8 = l * 8
            acc = [jnp.zeros((16,), jnp.float32) for _ in range(8)]
            for o in range(8):
                rowv = lanes + (l128 + o * 16)
                wv = w_v[pl.ds(l128 + o * 16, 16)]
                for f in range(8):
                    v = plsc.load_gather(rows_v, [rowv, fsplats[f]])
                    acc[f] = acc[f] + wv * v
            for f in range(8):
                val = acc[f] if cks[f // 2] is None else acc[f] * cks[f // 2]
                plsc.store_scatter(outc_v, [lanes, jnp.full((16,), lb8 + f, jnp.int32)],
                                   val)
            return c2
        lax.fori_loop(0, N_LEVELS, lvl_b, 0, unroll=False)

        pltpu.sync_copy(outc_v, out_hbm.at[pl.ds(wbase + ci * CHUNK, CHUNK), :])

    phase_a(0, idx0_v, w0_v)
    fire(idx0_v, rows0_v)

    def body2(j, carry):
        ci = j * 2
        phase_a(ci + 1, idx1_v, w1_v)
        fire(idx1_v, rows1_v)
        drain(idx0_v, rows0_v)
        phase_b(ci, w0_v, rows0_v)

        @pl.when(j < N_CHUNKS // 2 - 1)
        def _():
            phase_a(ci + 2, idx0_v, w0_v)
            fire(idx0_v, rows0_v)

        drain(idx1_v, rows1_v)
        phase_b(ci + 1, w1_v, rows1_v)
        return carry

    lax.fori_loop(0, N_CHUNKS // 2, body2, 0, unroll=False)


@functools.cache
def _build_sc_embed():
    mesh = plsc.VectorSubcoreMesh(core_axis_name="c", subcore_axis_name="s")
    return pl.kernel(
        _sc_embed_body,
        out_type=jax.ShapeDtypeStruct((N_PTS, N_LEVELS * N_DFT * F_PER_LEVEL),
                                      jnp.float32),
        mesh=mesh,
        compiler_params=pltpu.CompilerParams(needs_layout_passes=False,
                                             use_tc_tiling_on_sc=False),
        scratch_types=[
            pltpu.VMEM((P_PER_W,), jnp.float32),
            pltpu.VMEM((P_PER_W,), jnp.float32),
            pltpu.VMEM((P_PER_W,), jnp.float32),
            pltpu.VMEM((P_PER_W,), jnp.float32),
            pltpu.VMEM((P_PER_W,), jnp.float32),
            pltpu.VMEM((P_PER_W,), jnp.float32),
            pltpu.VMEM((N_LEVELS,), jnp.float32),
            pltpu.VMEM((N_LEVELS * 8 * CHUNK,), jnp.int32),
            pltpu.VMEM((N_LEVELS * 8 * CHUNK,), jnp.int32),
            pltpu.VMEM((N_LEVELS * 8 * CHUNK,), jnp.float32),
            pltpu.VMEM((N_LEVELS * 8 * CHUNK,), jnp.float32),
            pltpu.VMEM((N_LEVELS * 8 * CHUNK, N_DFT * F_PER_LEVEL), jnp.float32),
            pltpu.VMEM((N_LEVELS * 8 * CHUNK, N_DFT * F_PER_LEVEL), jnp.float32),
            pltpu.VMEM((CHUNK, N_LEVELS * N_DFT * F_PER_LEVEL), jnp.float32),
            pltpu.SemaphoreType.DMA,
        ],
    )


_BT = 8192


def _basis_body(t_ref, c1_ref, c2_ref, c3_ref):
    t = t_ref[...]
    c1 = jnp.cos(np.float32(np.pi) * t)
    c2 = 2.0 * c1 * c1 - 1.0
    c3 = 2.0 * c2 * c1 - c1
    c1_ref[...] = c1
    c2_ref[...] = c2
    c3_ref[...] = c3


@functools.cache
def _build_basis():
    return pl.pallas_call(
        _basis_body,
        grid=(N_PTS // _BT,),
        in_specs=[pl.BlockSpec((_BT,), lambda i: (i,))],
        out_specs=[pl.BlockSpec((_BT,), lambda i: (i,))] * 3,
        out_shape=[jax.ShapeDtypeStruct((N_PTS,), jnp.float32)] * 3,
    )


def _mlp_body(g_ref, w1_ref, b1_ref, w2_ref, b2_ref, w3_ref, b3_ref, o_ref):
    g = g_ref[...]
    srow = lax.broadcasted_iota(jnp.int32, (128, 32), 0)
    scol = lax.broadcasted_iota(jnp.int32, (128, 32), 1)
    sel = (scol == (srow // 8) * 2 + srow % 2).astype(jnp.float32)
    h32 = jnp.dot(g, sel, preferred_element_type=jnp.float32)
    h = jnp.maximum(jnp.dot(h32, w1_ref[...], preferred_element_type=jnp.float32)
                    + b1_ref[...], 0.0)
    h = jnp.maximum(jnp.dot(h, w2_ref[...], preferred_element_type=jnp.float32)
                    + b2_ref[...], 0.0)
    o_ref[...] = (jnp.dot(h, w3_ref[...], preferred_element_type=jnp.float32)
                  + b3_ref[...])


_BN = 1024


@functools.cache
def _build_mlp():
    d_in = N_LEVELS * N_DFT * F_PER_LEVEL
    return pl.pallas_call(
        _mlp_body,
        grid=(N_PTS // _BN,),
        in_specs=[
            pl.BlockSpec((_BN, d_in), lambda i: (i, 0)),
            pl.BlockSpec((2 * N_LEVELS, HIDDEN_), lambda i: (0, 0)),
            pl.BlockSpec((1, HIDDEN_), lambda i: (0, 0)),
            pl.BlockSpec((HIDDEN_, HIDDEN_), lambda i: (0, 0)),
            pl.BlockSpec((1, HIDDEN_), lambda i: (0, 0)),
            pl.BlockSpec((HIDDEN_, MLP_OUT_), lambda i: (0, 0)),
            pl.BlockSpec((1, MLP_OUT_), lambda i: (0, 0)),
        ],
        out_specs=pl.BlockSpec((_BN, MLP_OUT_), lambda i: (i, 0)),
        out_shape=jax.ShapeDtypeStruct((N_PTS, MLP_OUT_), jnp.float32),
    )


def kernel(x, t, tables, W1, b1, W2, b2, W3, b3):
    tab_flat = tables.reshape(N_LEVELS * TABLE_SIZE, N_DFT * F_PER_LEVEL)
    res = jnp.asarray(_RES_LIST, jnp.float32)
    c1, c2, c3 = _build_basis()(t)
    feats = _build_sc_embed()(x[:, 0], x[:, 1], x[:, 2], c1, c2, c3, res, tab_flat)
    return _build_mlp()(feats, W1, b1[None], W2, b2[None], W3, b3[None])


# speedup vs baseline: 1.2252x; 1.0400x over previous
import functools

import numpy as np
import jax
import jax.numpy as jnp
from jax import lax
from jax.experimental import pallas as pl
from jax.experimental.pallas import tpu as pltpu
from jax.experimental.pallas import tpu_sc as plsc

N_LEVELS = 16
F_PER_LEVEL = 2
LOG2_T = 16
TABLE_SIZE = 1 << LOG2_T
BASE_RES_ = 16
FINEST_RES_ = 512
N_DFT = 4
MLP_OUT_ = 16
N_PTS = 131072
HIDDEN_ = 64

_GROWTH = np.exp((np.log(FINEST_RES_) - np.log(BASE_RES_)) / (N_LEVELS - 1))
_RES_LIST = [float(np.floor(BASE_RES_ * _GROWTH ** l)) for l in range(N_LEVELS)]
_P1 = int(np.int32(np.uint32(2654435761)))
_P2 = int(np.int32(np.uint32(805459861)))

NC = 2
NS = 16
NW = NC * NS
P_PER_W = N_PTS // NW
CHUNK = 16
N_CHUNKS = P_PER_W // CHUNK


def _sc_embed_body(xr_hbm, yr_hbm, zr_hbm, c1_hbm, c2_hbm, c3_hbm, res_hbm,
                   tab_hbm, out_hbm,
                   xb_v, yb_v, zb_v, c1b_v, c2b_v, c3b_v, res_v,
                   idx0_v, idx1_v, w0_v, w1_v, rows0_v, rows1_v, outc_v, sem):
    cid = lax.axis_index("c")
    sid = lax.axis_index("s")
    wid = sid * NC + cid
    wbase = wid * P_PER_W

    pltpu.sync_copy(xr_hbm.at[pl.ds(wbase, P_PER_W)], xb_v)
    pltpu.sync_copy(yr_hbm.at[pl.ds(wbase, P_PER_W)], yb_v)
    pltpu.sync_copy(zr_hbm.at[pl.ds(wbase, P_PER_W)], zb_v)
    pltpu.sync_copy(c1_hbm.at[pl.ds(wbase, P_PER_W)], c1b_v)
    pltpu.sync_copy(c2_hbm.at[pl.ds(wbase, P_PER_W)], c2b_v)
    pltpu.sync_copy(c3_hbm.at[pl.ds(wbase, P_PER_W)], c3b_v)
    pltpu.sync_copy(res_hbm, res_v)

    lanes = lax.iota(jnp.int32, 16)
    lanes8 = lanes * 8

    def phase_a(ci, idx_v, w_v):
        base = ci * CHUNK
        px = xb_v[pl.ds(base, CHUNK)]
        py = yb_v[pl.ds(base, CHUNK)]
        pz = zb_v[pl.ds(base, CHUNK)]

        def lvl_a(l, c2):
            lsplat = jnp.full((16,), l, jnp.int32)
            r = plsc.load_gather(res_v, [lsplat])
            xs = px * r
            ys = py * r
            zs = pz * r
            xi = xs.astype(jnp.int32)
            yi = ys.astype(jnp.int32)
            zi = zs.astype(jnp.int32)
            wx1 = xs - xi.astype(jnp.float32)
            wy1 = ys - yi.astype(jnp.float32)
            wz1 = zs - zi.astype(jnp.float32)
            wx0 = 1.0 - wx1
            wy0 = 1.0 - wy1
            wz0 = 1.0 - wz1
            hx = (xi, xi + 1)
            hy = (yi * _P1, yi * _P1 + _P1)
            hz = (zi * _P2, zi * _P2 + _P2)
            wyz = (wy0 * wz0, wy0 * wz1, wy1 * wz0, wy1 * wz1)
            wx = (wx0, wx1)
            lbase = l * TABLE_SIZE
            l128 = l * 128
            for o in range(8):
                i, j, k = (o >> 2) & 1, (o >> 1) & 1, o & 1
                h = ((hx[i] ^ hy[j] ^ hz[k]) & 0xFFFF) + lbase
                idx_v[pl.ds(l128 + o * 16, 16)] = h
                w_v[pl.ds(l128 + o * 16, 16)] = wx[i] * wyz[2 * j + k]
            return c2
        lax.fori_loop(0, N_LEVELS, lvl_a, 0, unroll=False)

    def fire(idx_v, rows_v):
        pltpu.async_copy(tab_hbm.at[idx_v], rows_v, sem)

    def drain(idx_v, rows_v):
        pltpu.make_async_copy(tab_hbm.at[idx_v], rows_v, sem).wait()

    fsplats = [jnp.full((16,), f, jnp.int32) for f in range(8)]

    def phase_b(ci, w_v, rows_v):
        base = ci * CHUNK
        cks = (None, c1b_v[pl.ds(base, CHUNK)], c2b_v[pl.ds(base, CHUNK)],
               c3b_v[pl.ds(base, CHUNK)])

        def lvl_b(l, c2):
            l128 = l * 128
            lb8 = l * 8
            acc = [jnp.zeros((16,), jnp.float32) for _ in range(8)]
            for o in range(8):
                rowv = lanes + (l128 + o * 16)
                wv = w_v[pl.ds(l128 + o * 16, 16)]
                for f in range(8):
                    v = plsc.load_gather(rows_v, [rowv, fsplats[f]])
                    acc[f] = acc[f] + wv * v
            for f in range(8):
                val = acc[f] if cks[f // 2] is None else acc[f] * cks[f // 2]
                plsc.store_scatter(outc_v, [lanes, jnp.full((16,), lb8 + f, jnp.int32)],
                                   val)
            return c2
        lax.fori_loop(0, N_LEVELS, lvl_b, 0, unroll=False)

        pltpu.sync_copy(outc_v, out_hbm.at[pl.ds(wbase + ci * CHUNK, CHUNK), :])

    phase_a(0, idx0_v, w0_v)
    fire(idx0_v, rows0_v)

    def body2(j, carry):
        ci = j * 2
        phase_a(ci + 1, idx1_v, w1_v)
        fire(idx1_v, rows1_v)
        drain(idx0_v, rows0_v)
        phase_b(ci, w0_v, rows0_v)

        @pl.when(j < N_CHUNKS // 2 - 1)
        def _():
            phase_a(ci + 2, idx0_v, w0_v)
            fire(idx0_v, rows0_v)

        drain(idx1_v, rows1_v)
        phase_b(ci + 1, w1_v, rows1_v)
        return carry

    lax.fori_loop(0, N_CHUNKS // 2, body2, 0, unroll=False)


@functools.cache
def _build_sc_embed():
    mesh = plsc.VectorSubcoreMesh(core_axis_name="c", subcore_axis_name="s")
    return pl.kernel(
        _sc_embed_body,
        out_type=jax.ShapeDtypeStruct((N_PTS, N_LEVELS * N_DFT * F_PER_LEVEL),
                                      jnp.float32),
        mesh=mesh,
        compiler_params=pltpu.CompilerParams(needs_layout_passes=False,
                                             use_tc_tiling_on_sc=False),
        scratch_types=[
            pltpu.VMEM((P_PER_W,), jnp.float32),
            pltpu.VMEM((P_PER_W,), jnp.float32),
            pltpu.VMEM((P_PER_W,), jnp.float32),
            pltpu.VMEM((P_PER_W,), jnp.float32),
            pltpu.VMEM((P_PER_W,), jnp.float32),
            pltpu.VMEM((P_PER_W,), jnp.float32),
            pltpu.VMEM((N_LEVELS,), jnp.float32),
            pltpu.VMEM((N_LEVELS * 8 * CHUNK,), jnp.int32),
            pltpu.VMEM((N_LEVELS * 8 * CHUNK,), jnp.int32),
            pltpu.VMEM((N_LEVELS * 8 * CHUNK,), jnp.float32),
            pltpu.VMEM((N_LEVELS * 8 * CHUNK,), jnp.float32),
            pltpu.VMEM((N_LEVELS * 8 * CHUNK, N_DFT * F_PER_LEVEL), jnp.float32),
            pltpu.VMEM((N_LEVELS * 8 * CHUNK, N_DFT * F_PER_LEVEL), jnp.float32),
            pltpu.VMEM((CHUNK, N_LEVELS * N_DFT * F_PER_LEVEL), jnp.float32),
            pltpu.SemaphoreType.DMA,
        ],
    )


_BT = 8192


def _basis_body(t_ref, c1_ref, c2_ref, c3_ref):
    t = t_ref[...]
    c1 = jnp.cos(np.float32(np.pi) * t)
    c2 = 2.0 * c1 * c1 - 1.0
    c3 = 2.0 * c2 * c1 - c1
    c1_ref[...] = c1
    c2_ref[...] = c2
    c3_ref[...] = c3


@functools.cache
def _build_basis():
    return pl.pallas_call(
        _basis_body,
        grid=(N_PTS // _BT,),
        in_specs=[pl.BlockSpec((_BT,), lambda i: (i,))],
        out_specs=[pl.BlockSpec((_BT,), lambda i: (i,))] * 3,
        out_shape=[jax.ShapeDtypeStruct((N_PTS,), jnp.float32)] * 3,
    )


def _mlp_body(g_ref, w1_ref, b1_ref, w2_ref, b2_ref, w3_ref, b3_ref, o_ref):
    g = g_ref[...]
    srow = lax.broadcasted_iota(jnp.int32, (128, 32), 0)
    scol = lax.broadcasted_iota(jnp.int32, (128, 32), 1)
    sel = (scol == (srow // 8) * 2 + srow % 2).astype(jnp.float32)
    h32 = jnp.dot(g, sel, preferred_element_type=jnp.float32)
    h = jnp.maximum(jnp.dot(h32, w1_ref[...], preferred_element_type=jnp.float32)
                    + b1_ref[...], 0.0)
    h = jnp.maximum(jnp.dot(h, w2_ref[...], preferred_element_type=jnp.float32)
                    + b2_ref[...], 0.0)
    o_ref[...] = (jnp.dot(h, w3_ref[...], preferred_element_type=jnp.float32)
                  + b3_ref[...])


_BN = 4096


@functools.cache
def _build_mlp():
    d_in = N_LEVELS * N_DFT * F_PER_LEVEL
    return pl.pallas_call(
        _mlp_body,
        grid=(N_PTS // _BN,),
        in_specs=[
            pl.BlockSpec((_BN, d_in), lambda i: (i, 0)),
            pl.BlockSpec((2 * N_LEVELS, HIDDEN_), lambda i: (0, 0)),
            pl.BlockSpec((1, HIDDEN_), lambda i: (0, 0)),
            pl.BlockSpec((HIDDEN_, HIDDEN_), lambda i: (0, 0)),
            pl.BlockSpec((1, HIDDEN_), lambda i: (0, 0)),
            pl.BlockSpec((HIDDEN_, MLP_OUT_), lambda i: (0, 0)),
            pl.BlockSpec((1, MLP_OUT_), lambda i: (0, 0)),
        ],
        out_specs=pl.BlockSpec((_BN, MLP_OUT_), lambda i: (i, 0)),
        out_shape=jax.ShapeDtypeStruct((N_PTS, MLP_OUT_), jnp.float32),
    )


def kernel(x, t, tables, W1, b1, W2, b2, W3, b3):
    tab_flat = tables.reshape(N_LEVELS * TABLE_SIZE, N_DFT * F_PER_LEVEL)
    res = jnp.asarray(_RES_LIST, jnp.float32)
    c1, c2, c3 = _build_basis()(t)
    feats = _build_sc_embed()(x[:, 0], x[:, 1], x[:, 2], c1, c2, c3, res, tab_flat)
    return _build_mlp()(feats, W1, b1[None], W2, b2[None], W3, b3[None])


# speedup vs baseline: 1.2361x; 1.0089x over previous
import functools

import numpy as np
import jax
import jax.numpy as jnp
from jax import lax
from jax.experimental import pallas as pl
from jax.experimental.pallas import tpu as pltpu
from jax.experimental.pallas import tpu_sc as plsc

N_LEVELS = 16
F_PER_LEVEL = 2
LOG2_T = 16
TABLE_SIZE = 1 << LOG2_T
BASE_RES_ = 16
FINEST_RES_ = 512
N_DFT = 4
MLP_OUT_ = 16
N_PTS = 131072
HIDDEN_ = 64

_GROWTH = np.exp((np.log(FINEST_RES_) - np.log(BASE_RES_)) / (N_LEVELS - 1))
_RES_LIST = [float(np.floor(BASE_RES_ * _GROWTH ** l)) for l in range(N_LEVELS)]
_P1 = int(np.int32(np.uint32(2654435761)))
_P2 = int(np.int32(np.uint32(805459861)))

NC = 2
NS = 16
NW = NC * NS
P_PER_W = N_PTS // NW
CHUNK = 16
N_CHUNKS = P_PER_W // CHUNK


def _sc_embed_body(xr_hbm, yr_hbm, zr_hbm, c1_hbm, c2_hbm, c3_hbm, res_hbm,
                   tab_hbm, out_hbm,
                   xb_v, yb_v, zb_v, c1b_v, c2b_v, c3b_v, res_v,
                   idx0_v, idx1_v, w0_v, w1_v, rows0_v, rows1_v, outc_v, sem):
    cid = lax.axis_index("c")
    sid = lax.axis_index("s")
    wid = sid * NC + cid
    wbase = wid * P_PER_W

    pltpu.sync_copy(xr_hbm.at[pl.ds(wbase, P_PER_W)], xb_v)
    pltpu.sync_copy(yr_hbm.at[pl.ds(wbase, P_PER_W)], yb_v)
    pltpu.sync_copy(zr_hbm.at[pl.ds(wbase, P_PER_W)], zb_v)
    pltpu.sync_copy(c1_hbm.at[pl.ds(wbase, P_PER_W)], c1b_v)
    pltpu.sync_copy(c2_hbm.at[pl.ds(wbase, P_PER_W)], c2b_v)
    pltpu.sync_copy(c3_hbm.at[pl.ds(wbase, P_PER_W)], c3b_v)
    pltpu.sync_copy(res_hbm, res_v)

    lanes = lax.iota(jnp.int32, 16)
    lanes8 = lanes * 8

    def phase_a(ci, idx_v, w_v):
        base = ci * CHUNK
        px = xb_v[pl.ds(base, CHUNK)]
        py = yb_v[pl.ds(base, CHUNK)]
        pz = zb_v[pl.ds(base, CHUNK)]

        def lvl_a(l, c2):
            lsplat = jnp.full((16,), l, jnp.int32)
            r = plsc.load_gather(res_v, [lsplat])
            xs = px * r
            ys = py * r
            zs = pz * r
            xi = xs.astype(jnp.int32)
            yi = ys.astype(jnp.int32)
            zi = zs.astype(jnp.int32)
            wx1 = xs - xi.astype(jnp.float32)
            wy1 = ys - yi.astype(jnp.float32)
            wz1 = zs - zi.astype(jnp.float32)
            wx0 = 1.0 - wx1
            wy0 = 1.0 - wy1
            wz0 = 1.0 - wz1
            hx = (xi, xi + 1)
            hy = (yi * _P1, yi * _P1 + _P1)
            hz = (zi * _P2, zi * _P2 + _P2)
            wyz = (wy0 * wz0, wy0 * wz1, wy1 * wz0, wy1 * wz1)
            wx = (wx0, wx1)
            lbase = l * TABLE_SIZE
            l128 = l * 128
            for o in range(8):
                i, j, k = (o >> 2) & 1, (o >> 1) & 1, o & 1
                h = ((hx[i] ^ hy[j] ^ hz[k]) & 0xFFFF) + lbase
                idx_v[pl.ds(l128 + o * 16, 16)] = h
                w_v[pl.ds(l128 + o * 16, 16)] = wx[i] * wyz[2 * j + k]
            return c2
        lax.fori_loop(0, N_LEVELS, lvl_a, 0, unroll=False)

    def fire(idx_v, rows_v):
        pltpu.async_copy(tab_hbm.at[idx_v], rows_v, sem)

    def drain(idx_v, rows_v):
        pltpu.make_async_copy(tab_hbm.at[idx_v], rows_v, sem).wait()

    fsplats = [jnp.full((16,), f, jnp.int32) for f in range(8)]

    def phase_b(ci, w_v, rows_v):
        base = ci * CHUNK
        cks = (None, c1b_v[pl.ds(base, CHUNK)], c2b_v[pl.ds(base, CHUNK)],
               c3b_v[pl.ds(base, CHUNK)])

        def lvl_b(l, c2):
            l128 = l * 128
            lb8 = l * 8
            acc = [jnp.zeros((16,), jnp.float32) for _ in range(8)]
            for o in range(8):
                rowv = lanes + (l128 + o * 16)
                wv = w_v[pl.ds(l128 + o * 16, 16)]
                for f in range(8):
                    v = plsc.load_gather(rows_v, [rowv, fsplats[f]])
                    acc[f] = acc[f] + wv * v
            for f in range(8):
                val = acc[f] if cks[f // 2] is None else acc[f] * cks[f // 2]
                plsc.store_scatter(outc_v, [lanes, jnp.full((16,), lb8 + f, jnp.int32)],
                                   val)
            return c2
        lax.fori_loop(0, N_LEVELS, lvl_b, 0, unroll=False)

        pltpu.sync_copy(outc_v, out_hbm.at[pl.ds(wbase + ci * CHUNK, CHUNK), :])

    phase_a(0, idx0_v, w0_v)
    fire(idx0_v, rows0_v)

    def body2(j, carry):
        ci = j * 2
        phase_a(ci + 1, idx1_v, w1_v)
        fire(idx1_v, rows1_v)
        drain(idx0_v, rows0_v)
        phase_b(ci, w0_v, rows0_v)

        @pl.when(j < N_CHUNKS // 2 - 1)
        def _():
            phase_a(ci + 2, idx0_v, w0_v)
            fire(idx0_v, rows0_v)

        drain(idx1_v, rows1_v)
        phase_b(ci + 1, w1_v, rows1_v)
        return carry

    lax.fori_loop(0, N_CHUNKS // 2, body2, 0, unroll=False)


@functools.cache
def _build_sc_embed():
    mesh = plsc.VectorSubcoreMesh(core_axis_name="c", subcore_axis_name="s")
    return pl.kernel(
        _sc_embed_body,
        out_type=jax.ShapeDtypeStruct((N_PTS, N_LEVELS * N_DFT * F_PER_LEVEL),
                                      jnp.float32),
        mesh=mesh,
        compiler_params=pltpu.CompilerParams(needs_layout_passes=False,
                                             use_tc_tiling_on_sc=False),
        scratch_types=[
            pltpu.VMEM((P_PER_W,), jnp.float32),
            pltpu.VMEM((P_PER_W,), jnp.float32),
            pltpu.VMEM((P_PER_W,), jnp.float32),
            pltpu.VMEM((P_PER_W,), jnp.float32),
            pltpu.VMEM((P_PER_W,), jnp.float32),
            pltpu.VMEM((P_PER_W,), jnp.float32),
            pltpu.VMEM((N_LEVELS,), jnp.float32),
            pltpu.VMEM((N_LEVELS * 8 * CHUNK,), jnp.int32),
            pltpu.VMEM((N_LEVELS * 8 * CHUNK,), jnp.int32),
            pltpu.VMEM((N_LEVELS * 8 * CHUNK,), jnp.float32),
            pltpu.VMEM((N_LEVELS * 8 * CHUNK,), jnp.float32),
            pltpu.VMEM((N_LEVELS * 8 * CHUNK, N_DFT * F_PER_LEVEL), jnp.float32),
            pltpu.VMEM((N_LEVELS * 8 * CHUNK, N_DFT * F_PER_LEVEL), jnp.float32),
            pltpu.VMEM((CHUNK, N_LEVELS * N_DFT * F_PER_LEVEL), jnp.float32),
            pltpu.SemaphoreType.DMA,
        ],
    )


_BT = 8192


def _basis_body(t_ref, c1_ref, c2_ref, c3_ref):
    t = t_ref[...]
    c1 = jnp.cos(np.float32(np.pi) * t)
    c2 = 2.0 * c1 * c1 - 1.0
    c3 = 2.0 * c2 * c1 - c1
    c1_ref[...] = c1
    c2_ref[...] = c2
    c3_ref[...] = c3


@functools.cache
def _build_basis():
    return pl.pallas_call(
        _basis_body,
        grid=(N_PTS // _BT,),
        in_specs=[pl.BlockSpec((_BT,), lambda i: (i,))],
        out_specs=[pl.BlockSpec((_BT,), lambda i: (i,))] * 3,
        out_shape=[jax.ShapeDtypeStruct((N_PTS,), jnp.float32)] * 3,
    )


def _mlp_body(g_ref, w1_ref, b1_ref, w2_ref, b2_ref, w3_ref, b3_ref, o_ref):
    g = g_ref[...]
    srow = lax.broadcasted_iota(jnp.int32, (128, 32), 0)
    scol = lax.broadcasted_iota(jnp.int32, (128, 32), 1)
    sel = (scol == (srow // 8) * 2 + srow % 2).astype(jnp.float32)
    h32 = jnp.dot(g, sel, preferred_element_type=jnp.float32)
    h = jnp.maximum(jnp.dot(h32, w1_ref[...], preferred_element_type=jnp.float32)
                    + b1_ref[...], 0.0)
    h = jnp.maximum(jnp.dot(h, w2_ref[...], preferred_element_type=jnp.float32)
                    + b2_ref[...], 0.0)
    o_ref[...] = (jnp.dot(h, w3_ref[...], preferred_element_type=jnp.float32)
                  + b3_ref[...])


_BN = 8192


@functools.cache
def _build_mlp():
    d_in = N_LEVELS * N_DFT * F_PER_LEVEL
    return pl.pallas_call(
        _mlp_body,
        grid=(N_PTS // _BN,),
        in_specs=[
            pl.BlockSpec((_BN, d_in), lambda i: (i, 0)),
            pl.BlockSpec((2 * N_LEVELS, HIDDEN_), lambda i: (0, 0)),
            pl.BlockSpec((1, HIDDEN_), lambda i: (0, 0)),
            pl.BlockSpec((HIDDEN_, HIDDEN_), lambda i: (0, 0)),
            pl.BlockSpec((1, HIDDEN_), lambda i: (0, 0)),
            pl.BlockSpec((HIDDEN_, MLP_OUT_), lambda i: (0, 0)),
            pl.BlockSpec((1, MLP_OUT_), lambda i: (0, 0)),
        ],
        out_specs=pl.BlockSpec((_BN, MLP_OUT_), lambda i: (i, 0)),
        out_shape=jax.ShapeDtypeStruct((N_PTS, MLP_OUT_), jnp.float32),
    )


def kernel(x, t, tables, W1, b1, W2, b2, W3, b3):
    tab_flat = tables.reshape(N_LEVELS * TABLE_SIZE, N_DFT * F_PER_LEVEL)
    res = jnp.asarray(_RES_LIST, jnp.float32)
    c1, c2, c3 = _build_basis()(t)
    feats = _build_sc_embed()(x[:, 0], x[:, 1], x[:, 2], c1, c2, c3, res, tab_flat)
    return _build_mlp()(feats, W1, b1[None], W2, b2[None], W3, b3[None])
